# baseline (speedup 1.0000x reference)
import functools

import jax
import jax.numpy as jnp
from jax import lax
from jax.experimental import pallas as pl
from jax.experimental.pallas import tpu as pltpu
from jax.experimental.pallas import tpu_sc as plsc

_B, _S, _D = 4, 8192, 256
_N = _B * _S
_NC, _NS = 2, 16
_NW = _NC * _NS
_ROWS_PER_W = _N // _NW
_WPB = _S // _ROWS_PER_W
_CHUNK = 64
_NCHUNK = _ROWS_PER_W // _CHUNK
_NROWS_SLOT = 2
_NX_SLOT = 5
_LANES = 16
_DV = _D // _LANES


def _sc_add_gather(x, idx, table):
    mesh = plsc.VectorSubcoreMesh(core_axis_name="c", subcore_axis_name="s")

    scratch = [pltpu.VMEM((_ROWS_PER_W,), jnp.int32)]
    for _ in range(_NROWS_SLOT):
        scratch += [
            pltpu.VMEM((_CHUNK, _D), jnp.float32),
            pltpu.SemaphoreType.DMA,
        ]
    for _ in range(_NX_SLOT):
        scratch += [
            pltpu.VMEM((_CHUNK, _D), jnp.float32),
            pltpu.SemaphoreType.DMA,
            pltpu.SemaphoreType.DMA,
        ]

    @functools.partial(
        pl.kernel,
        mesh=mesh,
        out_type=jax.ShapeDtypeStruct((_B, _S, _D), jnp.float32),
        scratch_types=scratch,
    )
    def k(x_hbm, idx_hbm, tab_hbm, out_hbm, idx_v, *args):
        cid = lax.axis_index("c")
        sid = lax.axis_index("s")
        wid = sid * _NC + cid
        b = wid // _WPB
        s_base = (wid % _WPB) * _ROWS_PER_W

        rows_slots = [args[2 * g : 2 * g + 2] for g in range(_NROWS_SLOT)]
        xa = args[2 * _NROWS_SLOT :]
        x_slots = [xa[3 * g : 3 * g + 3] for g in range(_NX_SLOT)]

        pltpu.sync_copy(idx_hbm.at[b, pl.ds(s_base, _ROWS_PER_W)], idx_v)

        def start_gather(c):
            rows_v, sg = rows_slots[c % _NROWS_SLOT]
            return pltpu.async_copy(
                tab_hbm.at[idx_v.at[pl.ds(c * _CHUNK, _CHUNK)]], rows_v, sg)

        def start_xin(c):
            x_v, sx, _ = x_slots[c % _NX_SLOT]
            s0 = s_base + c * _CHUNK
            return pltpu.async_copy(x_hbm.at[b, pl.ds(s0, _CHUNK)], x_v, sx)

        def start_out(c):
            x_v, _, so = x_slots[c % _NX_SLOT]
            s0 = s_base + c * _CHUNK
            return pltpu.async_copy(x_v, out_hbm.at[b, pl.ds(s0, _CHUNK)], so)

        gather_d = {c: start_gather(c) for c in range(2)}
        xin_d = {c: start_xin(c) for c in range(2)}
        out_d = {}

        for c in range(_NCHUNK):
            rows_v, _ = rows_slots[c % _NROWS_SLOT]
            x_v, _, _ = x_slots[c % _NX_SLOT]
            gather_d.pop(c).wait()
            xin_d.pop(c).wait()

            def add_row(i, _, x_v=x_v, rows_v=rows_v):
                for j in range(_DV):
                    sl = pl.ds(j * _LANES, _LANES)
                    plsc.addupdate(x_v.at[i, sl], rows_v[i, sl])
                return 0

            lax.fori_loop(0, _CHUNK, add_row, 0)

            out_d[c] = start_out(c)
            if c + 2 < _NCHUNK:
                gather_d[c + 2] = start_gather(c + 2)
                if c - 3 >= 0:
                    out_d.pop(c - 3).wait()
                xin_d[c + 2] = start_xin(c + 2)

        for c in sorted(out_d):
            out_d.pop(c).wait()

    return k(x, idx, table)


def kernel(x, metonic_idx, phase_map):
    return _sc_add_gather(x, metonic_idx.astype(jnp.int32), phase_map)
